# baseline (speedup 1.0000x reference)
import jax
import jax.numpy as jnp
from jax import lax
from jax.experimental import pallas as pl
from jax.experimental.pallas import tpu as pltpu
from jax.experimental.pallas import tpu_sc as plsc

N = 10000
E = 320000
D = 128
H = 64
W = 128
NC = 2
NS = 16
NPAD = 10240
ROWS_PER_TILE = NPAD // NS
PB = 160
NB = NS * PB
CHUNK = 16
DUMMY = NPAD - 1


def _pre_body(x_ref, w1_ref, b1_ref, w2_ref, b2_ref,
              wf_ref, bf_ref, wb_ref, bb_ref,
              y_ref, mf_ref, mb_ref):
    x = x_ref[...]
    h = jnp.maximum(jnp.dot(x, w1_ref[...],
                            preferred_element_type=jnp.float32) + b1_ref[...], 0.0)
    y = jnp.dot(h, w2_ref[...], preferred_element_type=jnp.float32) + b2_ref[...]
    y_ref[...] = y
    mf = jnp.maximum(jnp.dot(y, wf_ref[...],
                             preferred_element_type=jnp.float32) + bf_ref[...], 0.0)
    mb = jnp.maximum(jnp.dot(y, wb_ref[...],
                             preferred_element_type=jnp.float32) + bb_ref[...], 0.0)
    rows = mf.shape[0]
    pad = jnp.concatenate(
        [jnp.ones((rows, 1), jnp.float32), jnp.zeros((rows, W - H - 1), jnp.float32)],
        axis=1)
    mf_ref[...] = jnp.concatenate([mf, pad], axis=1)
    mb_ref[...] = jnp.concatenate([mb, pad], axis=1)


def _dense_pre(x, W1, b1, W2, b2, Wf_pre, bf_pre, Wb_pre, bb_pre):
    blk = 1000
    grid = N // blk
    full = lambda shape: pl.BlockSpec(shape, lambda i: (0,) * len(shape))
    return pl.pallas_call(
        _pre_body,
        grid=(grid,),
        in_specs=[
            pl.BlockSpec((blk, D), lambda i: (i, 0)),
            full((D, D)), full((D,)), full((D, H)), full((H,)),
            full((H, H)), full((H,)), full((H, H)), full((H,)),
        ],
        out_specs=[
            pl.BlockSpec((blk, H), lambda i: (i, 0)),
            pl.BlockSpec((blk, W), lambda i: (i, 0)),
            pl.BlockSpec((blk, W), lambda i: (i, 0)),
        ],
        out_shape=[
            jax.ShapeDtypeStruct((N, H), jnp.float32),
            jax.ShapeDtypeStruct((NPAD, W), jnp.float32),
            jax.ShapeDtypeStruct((NPAD, W), jnp.float32),
        ],
    )(x, W1, b1, W2, b2, Wf_pre, bf_pre, Wb_pre, bb_pre)


def _sc_body(mf_hbm, mb_hbm, src_hbm, dst_hbm, zero_hbm,
             zf_out, zb_out,
             gbuf, sbuf, rows0, rows1,
             z_sh,
             gsem0, gsem1, ssem0, ssem1):
    c = lax.axis_index("c")
    s = lax.axis_index("s")
    rows = [rows0, rows1]
    gsem = [gsem0, gsem1]
    ssem = [ssem0, ssem1]

    row0 = s * ROWS_PER_TILE
    pltpu.sync_copy(zero_hbm, z_sh.at[pl.ds(row0, ROWS_PER_TILE)])
    plsc.subcore_barrier()

    t0 = s * PB

    def drain_scatter(k):
        pltpu.make_async_copy(zero_hbm.at[pl.ds(0, 128)], rows[k],
                              ssem[k]).wait()

    def edge_loop(m_hbm, gsrc_hbm, ssrc_hbm):
        def chunk_body(ci, carry):
            base = t0 + ci * CHUNK
            pltpu.sync_copy(gsrc_hbm.at[pl.ds(base, CHUNK)], gbuf)
            pltpu.sync_copy(ssrc_hbm.at[pl.ds(base, CHUNK)], sbuf)

            def group(h, carry2):
                pltpu.async_copy(
                    m_hbm.at[gbuf.at[h]], rows[0], gsem[0]).wait()
                pltpu.async_copy(rows[0], z_sh.at[sbuf.at[h]],
                                 ssem[0], add=True).wait()
                return carry2
            lax.fori_loop(0, CHUNK, group, 0)
            return carry
        lax.fori_loop(0, PB // CHUNK, chunk_body, 0)

    @pl.when(c == 0)
    def _():
        edge_loop(mf_hbm, src_hbm, dst_hbm)

    @pl.when(c == 1)
    def _():
        edge_loop(mb_hbm, dst_hbm, src_hbm)

    plsc.subcore_barrier()

    @pl.when(c == 0)
    def _():
        pltpu.sync_copy(z_sh.at[pl.ds(row0, ROWS_PER_TILE)],
                        zf_out.at[pl.ds(row0, ROWS_PER_TILE)])

    @pl.when(c == 1)
    def _():
        pltpu.sync_copy(z_sh.at[pl.ds(row0, ROWS_PER_TILE)],
                        zb_out.at[pl.ds(row0, ROWS_PER_TILE)])


def _sc_scatter(mf, mb, src2, dst2, zero):
    mesh = plsc.VectorSubcoreMesh(core_axis_name="c", subcore_axis_name="s")
    kern = pl.kernel(
        _sc_body,
        out_type=(
            jax.ShapeDtypeStruct((NPAD, W), jnp.float32),
            jax.ShapeDtypeStruct((NPAD, W), jnp.float32),
        ),
        mesh=mesh,
        scratch_types=[
            pltpu.VMEM((CHUNK, 128), jnp.int32),
            pltpu.VMEM((CHUNK, 128), jnp.int32),
            pltpu.VMEM((128, W), jnp.float32),
            pltpu.VMEM((128, W), jnp.float32),
            pltpu.VMEM_SHARED((NPAD, W), jnp.float32),
            pltpu.SemaphoreType.DMA, pltpu.SemaphoreType.DMA,
            pltpu.SemaphoreType.DMA, pltpu.SemaphoreType.DMA,
        ],
    )
    return kern(mf, mb, src2, dst2, zero)


def _post_body(y_ref, zf_ref, zb_ref, wf_ref, bf_ref, wb_ref, bb_ref, out_ref):
    y = y_ref[...]
    af = zf_ref[...]
    ab = zb_ref[...]
    zf = af[:, :H] / jnp.maximum(af[:, H:H + 1], 1.0)
    zb = ab[:, :H] / jnp.maximum(ab[:, H:H + 1], 1.0)
    hf = jnp.maximum(jnp.dot(zf, wf_ref[...],
                             preferred_element_type=jnp.float32) + bf_ref[...], 0.0) + y
    hb = jnp.maximum(jnp.dot(zb, wb_ref[...],
                             preferred_element_type=jnp.float32) + bb_ref[...], 0.0) + y
    out_ref[...] = jnp.concatenate([hf, hb], axis=1)


def _dense_post(y, zf, zb, Wf_upd, bf_upd, Wb_upd, bb_upd):
    blk = 1000
    grid = N // blk
    full = lambda shape: pl.BlockSpec(shape, lambda i: (0,) * len(shape))
    return pl.pallas_call(
        _post_body,
        grid=(grid,),
        in_specs=[
            pl.BlockSpec((blk, H), lambda i: (i, 0)),
            pl.BlockSpec((blk, W), lambda i: (i, 0)),
            pl.BlockSpec((blk, W), lambda i: (i, 0)),
            full((H, H)), full((H,)), full((H, H)), full((H,)),
        ],
        out_specs=pl.BlockSpec((blk, 2 * H), lambda i: (i, 0)),
        out_shape=jax.ShapeDtypeStruct((N, 2 * H), jnp.float32),
    )(y, zf, zb, Wf_upd, bf_upd, Wb_upd, bb_upd)


@jax.jit
def kernel(x, edge_index, W1, b1, W2, b2,
           Wf_pre, bf_pre, Wf_upd, bf_upd,
           Wb_pre, bb_pre, Wb_upd, bb_upd):
    y, mf, mb = _dense_pre(x, W1, b1, W2, b2, Wf_pre, bf_pre, Wb_pre, bb_pre)
    pad = jnp.full((NB * 128 - E,), DUMMY, edge_index.dtype)
    src2 = jnp.concatenate([edge_index[0], pad]).reshape(NB, 128)
    dst2 = jnp.concatenate([edge_index[1], pad]).reshape(NB, 128)
    zero = jnp.zeros((ROWS_PER_TILE, W), jnp.float32)
    zf, zb = _sc_scatter(mf, mb, src2, dst2, zero)
    return _dense_post(y, zf, zb, Wf_upd, bf_upd, Wb_upd, bb_upd)


# speedup vs baseline: 1.8726x; 1.8726x over previous
import jax
import jax.numpy as jnp
from jax import lax
from jax.experimental import pallas as pl
from jax.experimental.pallas import tpu as pltpu
from jax.experimental.pallas import tpu_sc as plsc

N = 10000
E = 320000
D = 128
H = 64
W = 128
NC = 2
NS = 16
NPAD = 10240
ROWS_PER_TILE = NPAD // NS
PB = 160
NB = NS * PB
CHUNK = 16
DUMMY = NPAD - 1


def _pre_body(x_ref, w1_ref, b1_ref, w2_ref, b2_ref,
              wf_ref, bf_ref, wb_ref, bb_ref,
              y_ref, mf_ref, mb_ref):
    x = x_ref[...]
    h = jnp.maximum(jnp.dot(x, w1_ref[...],
                            preferred_element_type=jnp.float32) + b1_ref[...], 0.0)
    y = jnp.dot(h, w2_ref[...], preferred_element_type=jnp.float32) + b2_ref[...]
    y_ref[...] = y
    mf = jnp.maximum(jnp.dot(y, wf_ref[...],
                             preferred_element_type=jnp.float32) + bf_ref[...], 0.0)
    mb = jnp.maximum(jnp.dot(y, wb_ref[...],
                             preferred_element_type=jnp.float32) + bb_ref[...], 0.0)
    rows = mf.shape[0]
    pad = jnp.concatenate(
        [jnp.ones((rows, 1), jnp.float32), jnp.zeros((rows, W - H - 1), jnp.float32)],
        axis=1)
    mf_ref[...] = jnp.concatenate([mf, pad], axis=1)
    mb_ref[...] = jnp.concatenate([mb, pad], axis=1)


def _dense_pre(x, W1, b1, W2, b2, Wf_pre, bf_pre, Wb_pre, bb_pre):
    blk = 1000
    grid = N // blk
    full = lambda shape: pl.BlockSpec(shape, lambda i: (0,) * len(shape))
    return pl.pallas_call(
        _pre_body,
        grid=(grid,),
        in_specs=[
            pl.BlockSpec((blk, D), lambda i: (i, 0)),
            full((D, D)), full((D,)), full((D, H)), full((H,)),
            full((H, H)), full((H,)), full((H, H)), full((H,)),
        ],
        out_specs=[
            pl.BlockSpec((blk, H), lambda i: (i, 0)),
            pl.BlockSpec((blk, W), lambda i: (i, 0)),
            pl.BlockSpec((blk, W), lambda i: (i, 0)),
        ],
        out_shape=[
            jax.ShapeDtypeStruct((N, H), jnp.float32),
            jax.ShapeDtypeStruct((NPAD, W), jnp.float32),
            jax.ShapeDtypeStruct((NPAD, W), jnp.float32),
        ],
    )(x, W1, b1, W2, b2, Wf_pre, bf_pre, Wb_pre, bb_pre)


def _sc_body(mf_hbm, mb_hbm, src_hbm, dst_hbm, zero_hbm,
             zf_out, zb_out,
             gbuf, sbuf, rows0, rows1,
             z_sh,
             gsem0, gsem1, ssem0, ssem1):
    c = lax.axis_index("c")
    s = lax.axis_index("s")
    rows = [rows0, rows1]
    gsem = [gsem0, gsem1]
    ssem = [ssem0, ssem1]

    row0 = s * ROWS_PER_TILE
    pltpu.sync_copy(zero_hbm, z_sh.at[pl.ds(row0, ROWS_PER_TILE)])
    plsc.subcore_barrier()

    t0 = s * PB

    def drain_scatter(k):
        pltpu.make_async_copy(zero_hbm.at[pl.ds(0, 128)], rows[k],
                              ssem[k]).wait()

    def edge_loop(m_hbm, gsrc_hbm, ssrc_hbm):
        def chunk_body(ci, carry):
            base = t0 + ci * CHUNK
            pltpu.sync_copy(gsrc_hbm.at[pl.ds(base, CHUNK)], gbuf)
            pltpu.sync_copy(ssrc_hbm.at[pl.ds(base, CHUNK)], sbuf)

            def group(h, carry2):
                pltpu.async_copy(
                    m_hbm.at[gbuf.at[h]], rows[0], gsem[0]).wait()
                pltpu.async_copy(rows[0], z_sh.at[sbuf.at[h]],
                                 ssem[0], add=True).wait()
                return carry2
            lax.fori_loop(0, CHUNK, group, 0)
            return carry
        lax.fori_loop(0, PB // CHUNK, chunk_body, 0)

    @pl.when(c == 0)
    def _():
        edge_loop(mf_hbm, src_hbm, dst_hbm)

    @pl.when(c == 1)
    def _():
        edge_loop(mb_hbm, dst_hbm, src_hbm)

    plsc.subcore_barrier()

    @pl.when(c == 0)
    def _():
        pltpu.sync_copy(z_sh.at[pl.ds(row0, ROWS_PER_TILE)],
                        zf_out.at[pl.ds(row0, ROWS_PER_TILE)])

    @pl.when(c == 1)
    def _():
        pltpu.sync_copy(z_sh.at[pl.ds(row0, ROWS_PER_TILE)],
                        zb_out.at[pl.ds(row0, ROWS_PER_TILE)])


def _sc_scatter(mf, mb, src2, dst2, zero):
    mesh = plsc.VectorSubcoreMesh(core_axis_name="c", subcore_axis_name="s")
    kern = pl.kernel(
        _sc_body,
        out_type=(
            jax.ShapeDtypeStruct((NPAD, W), jnp.float32),
            jax.ShapeDtypeStruct((NPAD, W), jnp.float32),
        ),
        mesh=mesh,
        scratch_types=[
            pltpu.VMEM((CHUNK, 128), jnp.int32),
            pltpu.VMEM((CHUNK, 128), jnp.int32),
            pltpu.VMEM((128, W), jnp.float32),
            pltpu.VMEM((128, W), jnp.float32),
            pltpu.VMEM_SHARED((NPAD, W), jnp.float32),
            pltpu.SemaphoreType.DMA, pltpu.SemaphoreType.DMA,
            pltpu.SemaphoreType.DMA, pltpu.SemaphoreType.DMA,
        ],
    )
    return kern(mf, mb, src2, dst2, zero)


def _post_body(y_ref, zf_ref, zb_ref, wf_ref, bf_ref, wb_ref, bb_ref, out_ref):
    y = y_ref[...]
    af = zf_ref[...]
    ab = zb_ref[...]
    zf = af[:, :H] / jnp.maximum(af[:, H:H + 1], 1.0)
    zb = ab[:, :H] / jnp.maximum(ab[:, H:H + 1], 1.0)
    hf = jnp.maximum(jnp.dot(zf, wf_ref[...],
                             preferred_element_type=jnp.float32) + bf_ref[...], 0.0) + y
    hb = jnp.maximum(jnp.dot(zb, wb_ref[...],
                             preferred_element_type=jnp.float32) + bb_ref[...], 0.0) + y
    out_ref[...] = jnp.concatenate([hf, hb], axis=1)


def _dense_post(y, zf, zb, Wf_upd, bf_upd, Wb_upd, bb_upd):
    blk = 1000
    grid = N // blk
    full = lambda shape: pl.BlockSpec(shape, lambda i: (0,) * len(shape))
    return pl.pallas_call(
        _post_body,
        grid=(grid,),
        in_specs=[
            pl.BlockSpec((blk, H), lambda i: (i, 0)),
            pl.BlockSpec((blk, W), lambda i: (i, 0)),
            pl.BlockSpec((blk, W), lambda i: (i, 0)),
            full((H, H)), full((H,)), full((H, H)), full((H,)),
        ],
        out_specs=pl.BlockSpec((blk, 2 * H), lambda i: (i, 0)),
        out_shape=jax.ShapeDtypeStruct((N, 2 * H), jnp.float32),
    )(y, zf, zb, Wf_upd, bf_upd, Wb_upd, bb_upd)


@jax.jit
def kernel(x, edge_index, W1, b1, W2, b2,
           Wf_pre, bf_pre, Wf_upd, bf_upd,
           Wb_pre, bb_pre, Wb_upd, bb_upd):
    y, mf, mb = _dense_pre(x, W1, b1, W2, b2, Wf_pre, bf_pre, Wb_pre, bb_pre)
    pad = N + jnp.arange(NB * 128 - E, dtype=edge_index.dtype) % (NPAD - N)
    src2 = jnp.concatenate([edge_index[0], pad]).reshape(NB, 128)
    dst2 = jnp.concatenate([edge_index[1], pad]).reshape(NB, 128)
    zero = jnp.zeros((ROWS_PER_TILE, W), jnp.float32)
    zf, zb = _sc_scatter(mf, mb, src2, dst2, zero)
    return _dense_post(y, zf, zb, Wf_upd, bf_upd, Wb_upd, bb_upd)


# speedup vs baseline: 2.1786x; 1.1634x over previous
import jax
import jax.numpy as jnp
from jax import lax
from jax.experimental import pallas as pl
from jax.experimental.pallas import tpu as pltpu
from jax.experimental.pallas import tpu_sc as plsc

N = 10000
E = 320000
D = 128
H = 64
W = 128
NC = 2
NS = 16
NPAD = 10240
ROWS_PER_TILE = NPAD // NS
PB = 160
NB = NS * PB
CHUNK = 16
DUMMY = NPAD - 1


def _pre_body(x_ref, w1_ref, b1_ref, w2_ref, b2_ref,
              wf_ref, bf_ref, wb_ref, bb_ref,
              y_ref, mf_ref, mb_ref):
    x = x_ref[...]
    h = jnp.maximum(jnp.dot(x, w1_ref[...],
                            preferred_element_type=jnp.float32) + b1_ref[...], 0.0)
    y = jnp.dot(h, w2_ref[...], preferred_element_type=jnp.float32) + b2_ref[...]
    y_ref[...] = y
    mf = jnp.maximum(jnp.dot(y, wf_ref[...],
                             preferred_element_type=jnp.float32) + bf_ref[...], 0.0)
    mb = jnp.maximum(jnp.dot(y, wb_ref[...],
                             preferred_element_type=jnp.float32) + bb_ref[...], 0.0)
    rows = mf.shape[0]
    pad = jnp.concatenate(
        [jnp.ones((rows, 1), jnp.float32), jnp.zeros((rows, W - H - 1), jnp.float32)],
        axis=1)
    mf_ref[...] = jnp.concatenate([mf, pad], axis=1)
    mb_ref[...] = jnp.concatenate([mb, pad], axis=1)


def _dense_pre(x, W1, b1, W2, b2, Wf_pre, bf_pre, Wb_pre, bb_pre):
    blk = 1000
    grid = N // blk
    full = lambda shape: pl.BlockSpec(shape, lambda i: (0,) * len(shape))
    return pl.pallas_call(
        _pre_body,
        grid=(grid,),
        in_specs=[
            pl.BlockSpec((blk, D), lambda i: (i, 0)),
            full((D, D)), full((D,)), full((D, H)), full((H,)),
            full((H, H)), full((H,)), full((H, H)), full((H,)),
        ],
        out_specs=[
            pl.BlockSpec((blk, H), lambda i: (i, 0)),
            pl.BlockSpec((blk, W), lambda i: (i, 0)),
            pl.BlockSpec((blk, W), lambda i: (i, 0)),
        ],
        out_shape=[
            jax.ShapeDtypeStruct((N, H), jnp.float32),
            jax.ShapeDtypeStruct((NPAD, W), jnp.float32),
            jax.ShapeDtypeStruct((NPAD, W), jnp.float32),
        ],
    )(x, W1, b1, W2, b2, Wf_pre, bf_pre, Wb_pre, bb_pre)


def _sc_body(mf_hbm, mb_hbm, src_hbm, dst_hbm, zero_hbm,
             zf_out, zb_out,
             gbuf, sbuf, rows0, rows1,
             z_sh,
             gsem0, gsem1, ssem0, ssem1):
    c = lax.axis_index("c")
    s = lax.axis_index("s")
    rows = [rows0, rows1]
    gsem = [gsem0, gsem1]
    ssem = [ssem0, ssem1]

    row0 = s * ROWS_PER_TILE
    pltpu.sync_copy(zero_hbm, z_sh.at[pl.ds(row0, ROWS_PER_TILE)])
    plsc.subcore_barrier()

    t0 = s * PB

    def drain_scatter(k):
        pltpu.make_async_copy(zero_hbm.at[pl.ds(0, 128)], rows[k],
                              ssem[k]).wait()

    def edge_loop(m_hbm, gsrc_hbm, ssrc_hbm):
        def chunk_body(ci, carry):
            for k in range(2):
                @pl.when(ci > 0)
                def _():
                    drain_scatter(k)
            base = t0 + ci * CHUNK
            pltpu.sync_copy(gsrc_hbm.at[pl.ds(base, CHUNK)], gbuf)
            pltpu.sync_copy(ssrc_hbm.at[pl.ds(base, CHUNK)], sbuf)

            def group(h, carry2):
                descs = []
                for k in range(2):
                    @pl.when(h > 0)
                    def _():
                        drain_scatter(k)
                    descs.append(pltpu.async_copy(
                        m_hbm.at[gbuf.at[2 * h + k]], rows[k], gsem[k]))
                for k in range(2):
                    descs[k].wait()
                    pltpu.async_copy(rows[k], z_sh.at[sbuf.at[2 * h + k]],
                                     ssem[k], add=True)
                return carry2
            lax.fori_loop(0, CHUNK // 2, group, 0)
            return carry
        lax.fori_loop(0, PB // CHUNK, chunk_body, 0)
        drain_scatter(0)
        drain_scatter(1)

    @pl.when(c == 0)
    def _():
        edge_loop(mf_hbm, src_hbm, dst_hbm)

    @pl.when(c == 1)
    def _():
        edge_loop(mb_hbm, dst_hbm, src_hbm)

    plsc.subcore_barrier()

    @pl.when(c == 0)
    def _():
        pltpu.sync_copy(z_sh.at[pl.ds(row0, ROWS_PER_TILE)],
                        zf_out.at[pl.ds(row0, ROWS_PER_TILE)])

    @pl.when(c == 1)
    def _():
        pltpu.sync_copy(z_sh.at[pl.ds(row0, ROWS_PER_TILE)],
                        zb_out.at[pl.ds(row0, ROWS_PER_TILE)])


def _sc_scatter(mf, mb, src2, dst2, zero):
    mesh = plsc.VectorSubcoreMesh(core_axis_name="c", subcore_axis_name="s")
    kern = pl.kernel(
        _sc_body,
        out_type=(
            jax.ShapeDtypeStruct((NPAD, W), jnp.float32),
            jax.ShapeDtypeStruct((NPAD, W), jnp.float32),
        ),
        mesh=mesh,
        scratch_types=[
            pltpu.VMEM((CHUNK, 128), jnp.int32),
            pltpu.VMEM((CHUNK, 128), jnp.int32),
            pltpu.VMEM((128, W), jnp.float32),
            pltpu.VMEM((128, W), jnp.float32),
            pltpu.VMEM_SHARED((NPAD, W), jnp.float32),
            pltpu.SemaphoreType.DMA, pltpu.SemaphoreType.DMA,
            pltpu.SemaphoreType.DMA, pltpu.SemaphoreType.DMA,
        ],
    )
    return kern(mf, mb, src2, dst2, zero)


def _post_body(y_ref, zf_ref, zb_ref, wf_ref, bf_ref, wb_ref, bb_ref, out_ref):
    y = y_ref[...]
    af = zf_ref[...]
    ab = zb_ref[...]
    zf = af[:, :H] / jnp.maximum(af[:, H:H + 1], 1.0)
    zb = ab[:, :H] / jnp.maximum(ab[:, H:H + 1], 1.0)
    hf = jnp.maximum(jnp.dot(zf, wf_ref[...],
                             preferred_element_type=jnp.float32) + bf_ref[...], 0.0) + y
    hb = jnp.maximum(jnp.dot(zb, wb_ref[...],
                             preferred_element_type=jnp.float32) + bb_ref[...], 0.0) + y
    out_ref[...] = jnp.concatenate([hf, hb], axis=1)


def _dense_post(y, zf, zb, Wf_upd, bf_upd, Wb_upd, bb_upd):
    blk = 1000
    grid = N // blk
    full = lambda shape: pl.BlockSpec(shape, lambda i: (0,) * len(shape))
    return pl.pallas_call(
        _post_body,
        grid=(grid,),
        in_specs=[
            pl.BlockSpec((blk, H), lambda i: (i, 0)),
            pl.BlockSpec((blk, W), lambda i: (i, 0)),
            pl.BlockSpec((blk, W), lambda i: (i, 0)),
            full((H, H)), full((H,)), full((H, H)), full((H,)),
        ],
        out_specs=pl.BlockSpec((blk, 2 * H), lambda i: (i, 0)),
        out_shape=jax.ShapeDtypeStruct((N, 2 * H), jnp.float32),
    )(y, zf, zb, Wf_upd, bf_upd, Wb_upd, bb_upd)


@jax.jit
def kernel(x, edge_index, W1, b1, W2, b2,
           Wf_pre, bf_pre, Wf_upd, bf_upd,
           Wb_pre, bb_pre, Wb_upd, bb_upd):
    y, mf, mb = _dense_pre(x, W1, b1, W2, b2, Wf_pre, bf_pre, Wb_pre, bb_pre)
    pad = N + jnp.arange(NB * 128 - E, dtype=edge_index.dtype) % (NPAD - N)
    src2 = jnp.concatenate([edge_index[0], pad]).reshape(NB, 128)
    dst2 = jnp.concatenate([edge_index[1], pad]).reshape(NB, 128)
    zero = jnp.zeros((ROWS_PER_TILE, W), jnp.float32)
    zf, zb = _sc_scatter(mf, mb, src2, dst2, zero)
    return _dense_post(y, zf, zb, Wf_upd, bf_upd, Wb_upd, bb_upd)


# speedup vs baseline: 2.7033x; 1.2409x over previous
import jax
import jax.numpy as jnp
from jax import lax
from jax.experimental import pallas as pl
from jax.experimental.pallas import tpu as pltpu
from jax.experimental.pallas import tpu_sc as plsc

N = 10000
E = 320000
D = 128
H = 64
W = 128
NC = 2
NS = 16
NPAD = 10240
ROWS_PER_TILE = NPAD // NS
PB = 160
NB = NS * PB
WZ = 80
CHUNK = 16
DUMMY = NPAD - 1


def _pre_body(x_ref, w1_ref, b1_ref, w2_ref, b2_ref,
              wf_ref, bf_ref, wb_ref, bb_ref,
              y_ref, mf_ref, mb_ref):
    x = x_ref[...]
    h = jnp.maximum(jnp.dot(x, w1_ref[...],
                            preferred_element_type=jnp.float32) + b1_ref[...], 0.0)
    y = jnp.dot(h, w2_ref[...], preferred_element_type=jnp.float32) + b2_ref[...]
    y_ref[...] = y
    mf = jnp.maximum(jnp.dot(y, wf_ref[...],
                             preferred_element_type=jnp.float32) + bf_ref[...], 0.0)
    mb = jnp.maximum(jnp.dot(y, wb_ref[...],
                             preferred_element_type=jnp.float32) + bb_ref[...], 0.0)
    rows = mf.shape[0]
    pad = jnp.concatenate(
        [jnp.ones((rows, 1), jnp.float32), jnp.zeros((rows, W - H - 1), jnp.float32)],
        axis=1)
    mf_ref[...] = jnp.concatenate([mf, pad], axis=1)
    mb_ref[...] = jnp.concatenate([mb, pad], axis=1)


def _dense_pre(x, W1, b1, W2, b2, Wf_pre, bf_pre, Wb_pre, bb_pre):
    blk = 1000
    grid = N // blk
    full = lambda shape: pl.BlockSpec(shape, lambda i: (0,) * len(shape))
    return pl.pallas_call(
        _pre_body,
        grid=(grid,),
        in_specs=[
            pl.BlockSpec((blk, D), lambda i: (i, 0)),
            full((D, D)), full((D,)), full((D, H)), full((H,)),
            full((H, H)), full((H,)), full((H, H)), full((H,)),
        ],
        out_specs=[
            pl.BlockSpec((blk, H), lambda i: (i, 0)),
            pl.BlockSpec((blk, W), lambda i: (i, 0)),
            pl.BlockSpec((blk, W), lambda i: (i, 0)),
        ],
        out_shape=[
            jax.ShapeDtypeStruct((N, H), jnp.float32),
            jax.ShapeDtypeStruct((NPAD, W), jnp.float32),
            jax.ShapeDtypeStruct((NPAD, W), jnp.float32),
        ],
    )(x, W1, b1, W2, b2, Wf_pre, bf_pre, Wb_pre, bb_pre)


def _sc_body(mf_hbm, mb_hbm, src_hbm, dst_hbm, zero_hbm,
             zf_out, zb_out,
             gbuf, sbuf, rows0, rows1,
             z_sh,
             gsem0, gsem1, ssem0, ssem1):
    c = lax.axis_index("c")
    s = lax.axis_index("s")
    rows = [rows0, rows1]
    gsem = [gsem0, gsem1]
    ssem = [ssem0, ssem1]

    row0 = s * ROWS_PER_TILE
    pltpu.sync_copy(zero_hbm, z_sh.at[pl.ds(row0, ROWS_PER_TILE)])
    plsc.subcore_barrier()

    t0 = s * PB

    def edge_loop(m_hbm, gsrc_hbm, ssrc_hbm):
        def chunk_body(ci, carry):
            base = t0 + ci * CHUNK
            pltpu.sync_copy(gsrc_hbm.at[pl.ds(base, CHUNK)], gbuf)
            pltpu.sync_copy(ssrc_hbm.at[pl.ds(base, CHUNK)], sbuf)

            gd = [None] * CHUNK
            sd = [None] * CHUNK
            for j in range(2):
                gd[j] = pltpu.async_copy(
                    m_hbm.at[gbuf.at[j]], rows[j], gsem[j])
            for j in range(CHUNK):
                k = j % 2
                gd[j].wait()
                sd[j] = pltpu.async_copy(
                    rows[k], z_sh.at[sbuf.at[j]], ssem[k], add=True)
                if j + 2 < CHUNK:
                    sd[j].wait()
                    gd[j + 2] = pltpu.async_copy(
                        m_hbm.at[gbuf.at[j + 2]], rows[k], gsem[k])
            sd[CHUNK - 2].wait()
            sd[CHUNK - 1].wait()
            return carry
        lax.fori_loop(0, PB // CHUNK, chunk_body, 0)

    @pl.when(c == 0)
    def _():
        edge_loop(mf_hbm, src_hbm, dst_hbm)

    @pl.when(c == 1)
    def _():
        edge_loop(mb_hbm, dst_hbm, src_hbm)

    plsc.subcore_barrier()

    @pl.when(c == 0)
    def _():
        pltpu.sync_copy(z_sh.at[pl.ds(row0, ROWS_PER_TILE)],
                        zf_out.at[pl.ds(row0, ROWS_PER_TILE)])

    @pl.when(c == 1)
    def _():
        pltpu.sync_copy(z_sh.at[pl.ds(row0, ROWS_PER_TILE)],
                        zb_out.at[pl.ds(row0, ROWS_PER_TILE)])


def _sc_scatter(mf, mb, src2, dst2, zero):
    mesh = plsc.VectorSubcoreMesh(core_axis_name="c", subcore_axis_name="s")
    kern = pl.kernel(
        _sc_body,
        out_type=(
            jax.ShapeDtypeStruct((NPAD, W), jnp.float32),
            jax.ShapeDtypeStruct((NPAD, W), jnp.float32),
        ),
        mesh=mesh,
        scratch_types=[
            pltpu.VMEM((CHUNK, 128), jnp.int32),
            pltpu.VMEM((CHUNK, 128), jnp.int32),
            pltpu.VMEM((128, W), jnp.float32),
            pltpu.VMEM((128, W), jnp.float32),
            pltpu.VMEM_SHARED((NPAD, W), jnp.float32),
            pltpu.SemaphoreType.DMA, pltpu.SemaphoreType.DMA,
            pltpu.SemaphoreType.DMA, pltpu.SemaphoreType.DMA,
        ],
    )
    return kern(mf, mb, src2, dst2, zero)


def _post_body(y_ref, zf_ref, zb_ref, wf_ref, bf_ref, wb_ref, bb_ref, out_ref):
    y = y_ref[...]
    af = zf_ref[...]
    ab = zb_ref[...]
    zf = af[:, :H] / jnp.maximum(af[:, H:H + 1], 1.0)
    zb = ab[:, :H] / jnp.maximum(ab[:, H:H + 1], 1.0)
    hf = jnp.maximum(jnp.dot(zf, wf_ref[...],
                             preferred_element_type=jnp.float32) + bf_ref[...], 0.0) + y
    hb = jnp.maximum(jnp.dot(zb, wb_ref[...],
                             preferred_element_type=jnp.float32) + bb_ref[...], 0.0) + y
    out_ref[...] = jnp.concatenate([hf, hb], axis=1)


def _dense_post(y, zf, zb, Wf_upd, bf_upd, Wb_upd, bb_upd):
    blk = 1000
    grid = N // blk
    full = lambda shape: pl.BlockSpec(shape, lambda i: (0,) * len(shape))
    return pl.pallas_call(
        _post_body,
        grid=(grid,),
        in_specs=[
            pl.BlockSpec((blk, H), lambda i: (i, 0)),
            pl.BlockSpec((blk, W), lambda i: (i, 0)),
            pl.BlockSpec((blk, W), lambda i: (i, 0)),
            full((H, H)), full((H,)), full((H, H)), full((H,)),
        ],
        out_specs=pl.BlockSpec((blk, 2 * H), lambda i: (i, 0)),
        out_shape=jax.ShapeDtypeStruct((N, 2 * H), jnp.float32),
    )(y, zf, zb, Wf_upd, bf_upd, Wb_upd, bb_upd)


@jax.jit
def kernel(x, edge_index, W1, b1, W2, b2,
           Wf_pre, bf_pre, Wf_upd, bf_upd,
           Wb_pre, bb_pre, Wb_upd, bb_upd):
    y, mf, mb = _dense_pre(x, W1, b1, W2, b2, Wf_pre, bf_pre, Wb_pre, bb_pre)
    pad = N + jnp.arange(NB * 128 - E, dtype=edge_index.dtype) % (NPAD - N)
    src2 = jnp.concatenate([edge_index[0], pad]).reshape(NB, 128)
    dst2 = jnp.concatenate([edge_index[1], pad]).reshape(NB, 128)
    zero = jnp.zeros((ROWS_PER_TILE, W), jnp.float32)
    zf, zb = _sc_scatter(mf, mb, src2, dst2, zero)
    return _dense_post(y, zf, zb, Wf_upd, bf_upd, Wb_upd, bb_upd)


# speedup vs baseline: 2.9251x; 1.0820x over previous
import jax
import jax.numpy as jnp
from jax import lax
from jax.experimental import pallas as pl
from jax.experimental.pallas import tpu as pltpu
from jax.experimental.pallas import tpu_sc as plsc

N = 10000
E = 320000
D = 128
H = 64
W = 128
NC = 2
NS = 16
NPAD = 10240
ROWS_PER_TILE = NPAD // NS
B = 64
PB = 320
NB = NS * PB
SLOTS = 4
CHUNK = 32
DUMMY = NPAD - 1


def _pre_body(x_ref, w1_ref, b1_ref, w2_ref, b2_ref,
              wf_ref, bf_ref, wb_ref, bb_ref,
              y_ref, mf_ref, mb_ref):
    x = x_ref[...]
    h = jnp.maximum(jnp.dot(x, w1_ref[...],
                            preferred_element_type=jnp.float32) + b1_ref[...], 0.0)
    y = jnp.dot(h, w2_ref[...], preferred_element_type=jnp.float32) + b2_ref[...]
    y_ref[...] = y
    mf = jnp.maximum(jnp.dot(y, wf_ref[...],
                             preferred_element_type=jnp.float32) + bf_ref[...], 0.0)
    mb = jnp.maximum(jnp.dot(y, wb_ref[...],
                             preferred_element_type=jnp.float32) + bb_ref[...], 0.0)
    rows = mf.shape[0]
    pad = jnp.concatenate(
        [jnp.ones((rows, 1), jnp.float32), jnp.zeros((rows, W - H - 1), jnp.float32)],
        axis=1)
    mf_ref[...] = jnp.concatenate([mf, pad], axis=1)
    mb_ref[...] = jnp.concatenate([mb, pad], axis=1)


def _dense_pre(x, W1, b1, W2, b2, Wf_pre, bf_pre, Wb_pre, bb_pre):
    blk = 1000
    grid = N // blk
    full = lambda shape: pl.BlockSpec(shape, lambda i: (0,) * len(shape))
    return pl.pallas_call(
        _pre_body,
        grid=(grid,),
        in_specs=[
            pl.BlockSpec((blk, D), lambda i: (i, 0)),
            full((D, D)), full((D,)), full((D, H)), full((H,)),
            full((H, H)), full((H,)), full((H, H)), full((H,)),
        ],
        out_specs=[
            pl.BlockSpec((blk, H), lambda i: (i, 0)),
            pl.BlockSpec((blk, W), lambda i: (i, 0)),
            pl.BlockSpec((blk, W), lambda i: (i, 0)),
        ],
        out_shape=[
            jax.ShapeDtypeStruct((N, H), jnp.float32),
            jax.ShapeDtypeStruct((NPAD, W), jnp.float32),
            jax.ShapeDtypeStruct((NPAD, W), jnp.float32),
        ],
    )(x, W1, b1, W2, b2, Wf_pre, bf_pre, Wb_pre, bb_pre)


def _sc_body(mf_hbm, mb_hbm, src_hbm, dst_hbm, zero_hbm,
             zf_out, zb_out,
             gbuf, sbuf, rows0, rows1, rows2, rows3,
             z_sh,
             gsem0, gsem1, gsem2, gsem3, ssem0, ssem1, ssem2, ssem3):
    c = lax.axis_index("c")
    s = lax.axis_index("s")
    rows = [rows0, rows1, rows2, rows3]
    gsem = [gsem0, gsem1, gsem2, gsem3]
    ssem = [ssem0, ssem1, ssem2, ssem3]

    row0 = s * ROWS_PER_TILE
    pltpu.sync_copy(zero_hbm, z_sh.at[pl.ds(row0, ROWS_PER_TILE)])
    plsc.subcore_barrier()

    t0 = s * PB

    def edge_loop(m_hbm, gsrc_hbm, ssrc_hbm):
        def chunk_body(ci, carry):
            base = t0 + ci * CHUNK
            pltpu.sync_copy(gsrc_hbm.at[pl.ds(base, CHUNK)], gbuf)
            pltpu.sync_copy(ssrc_hbm.at[pl.ds(base, CHUNK)], sbuf)

            gd = [None] * CHUNK
            sd = [None] * CHUNK
            for j in range(SLOTS):
                gd[j] = pltpu.async_copy(
                    m_hbm.at[gbuf.at[j]], rows[j], gsem[j])
            for j in range(CHUNK):
                k = j % SLOTS
                gd[j].wait()
                sd[j] = pltpu.async_copy(
                    rows[k], z_sh.at[sbuf.at[j]], ssem[k], add=True)
                if j + SLOTS < CHUNK:
                    sd[j].wait()
                    gd[j + SLOTS] = pltpu.async_copy(
                        m_hbm.at[gbuf.at[j + SLOTS]], rows[k], gsem[k])
            for j in range(CHUNK - SLOTS, CHUNK):
                sd[j].wait()
            return carry
        lax.fori_loop(0, PB // CHUNK, chunk_body, 0)

    @pl.when(c == 0)
    def _():
        edge_loop(mf_hbm, src_hbm, dst_hbm)

    @pl.when(c == 1)
    def _():
        edge_loop(mb_hbm, dst_hbm, src_hbm)

    plsc.subcore_barrier()

    @pl.when(c == 0)
    def _():
        pltpu.sync_copy(z_sh.at[pl.ds(row0, ROWS_PER_TILE)],
                        zf_out.at[pl.ds(row0, ROWS_PER_TILE)])

    @pl.when(c == 1)
    def _():
        pltpu.sync_copy(z_sh.at[pl.ds(row0, ROWS_PER_TILE)],
                        zb_out.at[pl.ds(row0, ROWS_PER_TILE)])


def _sc_scatter(mf, mb, src2, dst2, zero):
    mesh = plsc.VectorSubcoreMesh(core_axis_name="c", subcore_axis_name="s")
    kern = pl.kernel(
        _sc_body,
        out_type=(
            jax.ShapeDtypeStruct((NPAD, W), jnp.float32),
            jax.ShapeDtypeStruct((NPAD, W), jnp.float32),
        ),
        mesh=mesh,
        scratch_types=[
            pltpu.VMEM((CHUNK, B), jnp.int32),
            pltpu.VMEM((CHUNK, B), jnp.int32),
            pltpu.VMEM((B, W), jnp.float32),
            pltpu.VMEM((B, W), jnp.float32),
            pltpu.VMEM((B, W), jnp.float32),
            pltpu.VMEM((B, W), jnp.float32),
            pltpu.VMEM_SHARED((NPAD, W), jnp.float32),
            pltpu.SemaphoreType.DMA, pltpu.SemaphoreType.DMA,
            pltpu.SemaphoreType.DMA, pltpu.SemaphoreType.DMA,
            pltpu.SemaphoreType.DMA, pltpu.SemaphoreType.DMA,
            pltpu.SemaphoreType.DMA, pltpu.SemaphoreType.DMA,
        ],
    )
    return kern(mf, mb, src2, dst2, zero)


def _post_body(y_ref, zf_ref, zb_ref, wf_ref, bf_ref, wb_ref, bb_ref, out_ref):
    y = y_ref[...]
    af = zf_ref[...]
    ab = zb_ref[...]
    zf = af[:, :H] / jnp.maximum(af[:, H:H + 1], 1.0)
    zb = ab[:, :H] / jnp.maximum(ab[:, H:H + 1], 1.0)
    hf = jnp.maximum(jnp.dot(zf, wf_ref[...],
                             preferred_element_type=jnp.float32) + bf_ref[...], 0.0) + y
    hb = jnp.maximum(jnp.dot(zb, wb_ref[...],
                             preferred_element_type=jnp.float32) + bb_ref[...], 0.0) + y
    out_ref[...] = jnp.concatenate([hf, hb], axis=1)


def _dense_post(y, zf, zb, Wf_upd, bf_upd, Wb_upd, bb_upd):
    blk = 1000
    grid = N // blk
    full = lambda shape: pl.BlockSpec(shape, lambda i: (0,) * len(shape))
    return pl.pallas_call(
        _post_body,
        grid=(grid,),
        in_specs=[
            pl.BlockSpec((blk, H), lambda i: (i, 0)),
            pl.BlockSpec((blk, W), lambda i: (i, 0)),
            pl.BlockSpec((blk, W), lambda i: (i, 0)),
            full((H, H)), full((H,)), full((H, H)), full((H,)),
        ],
        out_specs=pl.BlockSpec((blk, 2 * H), lambda i: (i, 0)),
        out_shape=jax.ShapeDtypeStruct((N, 2 * H), jnp.float32),
    )(y, zf, zb, Wf_upd, bf_upd, Wb_upd, bb_upd)


@jax.jit
def kernel(x, edge_index, W1, b1, W2, b2,
           Wf_pre, bf_pre, Wf_upd, bf_upd,
           Wb_pre, bb_pre, Wb_upd, bb_upd):
    y, mf, mb = _dense_pre(x, W1, b1, W2, b2, Wf_pre, bf_pre, Wb_pre, bb_pre)
    pad = N + jnp.arange(NB * B - E, dtype=edge_index.dtype) % (NPAD - N)
    src2 = jnp.concatenate([edge_index[0], pad]).reshape(NB, B)
    dst2 = jnp.concatenate([edge_index[1], pad]).reshape(NB, B)
    zero = jnp.zeros((ROWS_PER_TILE, W), jnp.float32)
    zf, zb = _sc_scatter(mf, mb, src2, dst2, zero)
    return _dense_post(y, zf, zb, Wf_upd, bf_upd, Wb_upd, bb_upd)
